# Initial kernel scaffold; baseline (speedup 1.0000x reference)
import jax
import jax.numpy as jnp
from jax.experimental import pallas as pl


def kernel(src, dst, t, msg, x_pol, x_comp, memory, pol_W, pol_b, comp_W, comp_b, bn_gamma, bn_beta, bn_mean, bn_var, W1, b1, W2, b2, W3, b3):
    raise NotImplementedError("write your pallas kernel here")


# speedup vs baseline: 1.8430x; 1.8430x over previous
import functools

import jax
import jax.numpy as jnp
from jax import lax
from jax.experimental import pallas as pl
from jax.experimental.pallas import tpu as pltpu
from jax.experimental.pallas import tpu_sc as plsc

_B = 320000
_N = 10000
_D = 128
_NW = 32
_CPN = 157
_RPW = _CPN * 128
_RPAD = _NW * _RPW
_NBUF = 4
_BT = 512


def _pre_body(mem_ref, w_ref, t_ref):
    t_ref[...] = jnp.dot(mem_ref[...], w_ref[0],
                         preferred_element_type=jnp.float32)


def _precompute_table(memory, w_stack):
    nblk = 10
    rows = _N // nblk
    return pl.pallas_call(
        _pre_body,
        grid=(2 * nblk,),
        in_specs=[
            pl.BlockSpec((rows, _D), lambda j: (j % nblk, 0)),
            pl.BlockSpec((1, _D, _D), lambda j: (j // nblk, 0, 0)),
        ],
        out_specs=pl.BlockSpec((rows, _D), lambda j: (j, 0)),
        out_shape=jax.ShapeDtypeStruct((2 * _N, _D), jnp.float32),
        compiler_params=pltpu.CompilerParams(
            dimension_semantics=("arbitrary",)),
    )(memory, w_stack)


def _gather_body(idx_hbm, table_hbm, out_hbm, idx_v, rows_v, gsem, osem):
    c = lax.axis_index("c")
    s = lax.axis_index("s")
    wid = s * 2 + c
    base = wid * _RPW
    pltpu.sync_copy(idx_hbm.at[wid], idx_v)

    def gather(i, b):
        return pltpu.make_async_copy(
            table_hbm.at[idx_v.at[i]], rows_v.at[b], gsem.at[b])

    def write(i, b):
        return pltpu.make_async_copy(
            rows_v.at[b], out_hbm.at[pl.ds(base + i * 128, 128)], osem.at[b])

    for b in range(_NBUF):
        gather(b, b).start()

    def body(i, carry):
        b = i % _NBUF
        gather(i, b).wait()
        write(i, b).start()
        write(i, b).wait()
        gather(i + _NBUF, b).start()
        return carry

    lax.fori_loop(0, _CPN - _NBUF, body, 0)
    for k in range(_CPN - _NBUF, _CPN):
        gather(k, k % _NBUF).wait()
        write(k, k % _NBUF).start()
    for k in range(_CPN - _NBUF, _CPN):
        write(k, k % _NBUF).wait()


_gather_call = functools.partial(
    pl.kernel,
    out_type=jax.ShapeDtypeStruct((_RPAD, _D), jnp.float32),
    mesh=plsc.VectorSubcoreMesh(core_axis_name="c", subcore_axis_name="s"),
    scratch_types=[
        pltpu.VMEM((_CPN, 128), jnp.int32),
        pltpu.VMEM((_NBUF, 128, _D), jnp.float32),
        pltpu.SemaphoreType.DMA((_NBUF,)),
        pltpu.SemaphoreType.DMA((_NBUF,)),
    ],
)(_gather_body)


def _mlp_body(ga_ref, gc_ref, xp_ref, xc_ref, wp_ref, wc_ref, b1_ref,
              w2_ref, b2_ref, w3_ref, b3_ref, out_ref):
    z = ga_ref[...] + gc_ref[...] + b1_ref[...]
    z = z + jnp.dot(xp_ref[...], wp_ref[...],
                    preferred_element_type=jnp.float32)
    z = z + jnp.dot(xc_ref[...], wc_ref[...],
                    preferred_element_type=jnp.float32)
    h1 = jnp.maximum(z, 0.0)
    h2 = jnp.dot(h1, w2_ref[...], preferred_element_type=jnp.float32)
    h2 = jnp.maximum(h2 + b2_ref[...], 0.0)
    logit = jnp.dot(h2, w3_ref[...], preferred_element_type=jnp.float32)
    out_ref[...] = jax.nn.sigmoid(logit + b3_ref[...])


def _mlp_call(g, x_pol, x_comp, wp, wc, b1, w2, b2, w3, b3):
    nb = _B // _BT
    const = lambda i: (0, 0)
    return pl.pallas_call(
        _mlp_body,
        grid=(nb,),
        in_specs=[
            pl.BlockSpec((_BT, _D), lambda i: (i, 0)),
            pl.BlockSpec((_BT, _D), lambda i: (i + nb, 0)),
            pl.BlockSpec((_BT, _D), lambda i: (i, 0)),
            pl.BlockSpec((_BT, _D), lambda i: (i, 0)),
            pl.BlockSpec((_D, _D), const),
            pl.BlockSpec((_D, _D), const),
            pl.BlockSpec((1, _D), const),
            pl.BlockSpec((_D, 64), const),
            pl.BlockSpec((1, 64), const),
            pl.BlockSpec((64, 1), const),
            pl.BlockSpec((1, 1), const),
        ],
        out_specs=pl.BlockSpec((_BT, 1), lambda i: (i, 0)),
        out_shape=jax.ShapeDtypeStruct((_B, 1), jnp.float32),
        compiler_params=pltpu.CompilerParams(
            dimension_semantics=("parallel",)),
    )(g, g, x_pol, x_comp, wp, wc, b1, w2, b2, w3, b3)


def kernel(src, dst, t, msg, x_pol, x_comp, memory, pol_W, pol_b, comp_W,
           comp_b, bn_gamma, bn_beta, bn_mean, bn_var, W1, b1, W2, b2, W3,
           b3):
    del t, msg

    scale = bn_gamma * lax.rsqrt(bn_var + 1e-5)
    shift = bn_beta - bn_mean * scale
    W1f = W1 * scale[:, None]
    W1a, W1b = W1f[0:_D], W1f[_D:2 * _D]
    W1c, W1d = W1f[2 * _D:3 * _D], W1f[3 * _D:4 * _D]
    wp = pol_W @ W1b
    wc = comp_W @ W1d
    btot = (b1 + shift @ W1 + pol_b @ W1b + comp_b @ W1d).reshape(1, _D)

    table = _precompute_table(memory, jnp.stack([W1a, W1c]))

    idx = jnp.concatenate([
        src.astype(jnp.int32),
        dst.astype(jnp.int32) + _N,
        jnp.zeros((_RPAD - 2 * _B,), jnp.int32),
    ]).reshape(_NW, _CPN, 128)

    g = _gather_call(idx, table)

    return _mlp_call(g, x_pol, x_comp, wp, wc, btot,
                     W2, b2.reshape(1, 64), W3, b3.reshape(1, 1))
